# Initial kernel scaffold; baseline (speedup 1.0000x reference)
import jax
import jax.numpy as jnp
from jax.experimental import pallas as pl


def kernel(x, edge_index, edge_weight, W):
    raise NotImplementedError("write your pallas kernel here")


# speedup vs baseline: 4.0454x; 4.0454x over previous
import functools

import jax
import jax.numpy as jnp
from jax import lax
from jax.experimental import pallas as pl
from jax.experimental.pallas import tpu as pltpu
from jax.experimental.pallas import tpu_sc as plsc

N_NODES = 10000
D = 128
N_EDGES = 320000

NC = 2
NS = 16
NW = NC * NS
LANES = 16

K = 128
C = 79
E_W = C * K
E_PAD = NW * E_W
ROWS_PER_TILE = 624
TAIL_ROWS = N_NODES - NS * ROWS_PER_TILE
TAIL_OFF = NS * ROWS_PER_TILE


def _sc_aggregate(x, src, dst, w, zeros):
    mesh = plsc.VectorSubcoreMesh(
        core_axis_name="c", subcore_axis_name="s", num_cores=NC, num_subcores=NS
    )

    @functools.partial(
        pl.kernel,
        out_type=jax.ShapeDtypeStruct((NC, N_NODES, D), jnp.float32),
        mesh=mesh,
        scratch_types=[
            pltpu.VMEM_SHARED((N_NODES, D), jnp.float32),
            pltpu.VMEM((C, K), jnp.int32),
            pltpu.VMEM((C, K), jnp.int32),
            pltpu.VMEM((E_W,), jnp.float32),
            pltpu.VMEM((K, D), jnp.float32),
        ],
    )
    def agg(x_hbm, src_hbm, dst_hbm, w_hbm, zeros_hbm, part_hbm,
            acc, src_v, dst_v, w_v, rows_v):
        cid = lax.axis_index("c")
        sid = lax.axis_index("s")
        wid = sid * NC + cid

        pltpu.sync_copy(zeros_hbm.at[pl.ds(sid * ROWS_PER_TILE, ROWS_PER_TILE)],
                        acc.at[pl.ds(sid * ROWS_PER_TILE, ROWS_PER_TILE)])

        @pl.when(sid == 0)
        def _():
            pltpu.sync_copy(zeros_hbm.at[pl.ds(TAIL_OFF, TAIL_ROWS)],
                            acc.at[pl.ds(TAIL_OFF, TAIL_ROWS)])
        pltpu.sync_copy(src_hbm.at[wid], src_v)
        pltpu.sync_copy(dst_hbm.at[wid], dst_v)
        pltpu.sync_copy(w_hbm.at[wid], w_v)
        plsc.subcore_barrier()

        def chunk_body(c, carry):
            pltpu.sync_copy(x_hbm.at[src_v.at[c]], rows_v)
            base = c * K
            for eb in range(K // LANES):
                wvec = w_v[pl.ds(base + eb * LANES, LANES)]
                for j in range(LANES):
                    e = eb * LANES + j
                    wb = wvec[j]
                    for g in range(D // LANES):
                        sl = pl.ds(g * LANES, LANES)
                        rows_v[e, sl] = rows_v[e, sl] * wb
            pltpu.sync_copy(rows_v, acc.at[dst_v.at[c]], add=True)
            return carry

        lax.fori_loop(0, C, chunk_body, 0)

        plsc.subcore_barrier()
        pltpu.sync_copy(acc.at[pl.ds(sid * ROWS_PER_TILE, ROWS_PER_TILE)],
                        part_hbm.at[cid, pl.ds(sid * ROWS_PER_TILE, ROWS_PER_TILE)])

        @pl.when(sid == 0)
        def _():
            pltpu.sync_copy(acc.at[pl.ds(TAIL_OFF, TAIL_ROWS)],
                            part_hbm.at[cid, pl.ds(TAIL_OFF, TAIL_ROWS)])

    return agg(x, src, dst, w, zeros)


def _tc_finish(parts, W):
    R = 1000

    def body(p_ref, w_ref, o_ref):
        p = p_ref[0] + p_ref[1]
        y = jnp.dot(p, w_ref[...], preferred_element_type=jnp.float32)
        o_ref[...] = jnp.maximum(y, 0.0)

    return pl.pallas_call(
        body,
        grid=(N_NODES // R,),
        in_specs=[
            pl.BlockSpec((NC, R, D), lambda i: (0, i, 0)),
            pl.BlockSpec((D, D), lambda i: (0, 0)),
        ],
        out_specs=pl.BlockSpec((R, D), lambda i: (i, 0)),
        out_shape=jax.ShapeDtypeStruct((N_NODES, D), jnp.float32),
    )(parts, W)


def kernel(x, edge_index, edge_weight, W):
    pad = E_PAD - N_EDGES
    ei = jnp.concatenate(
        [edge_index, jnp.zeros((2, pad), edge_index.dtype)], axis=1)
    dst = ei[0].reshape(NW, C, K)
    src = ei[1].reshape(NW, C, K)
    w = jnp.concatenate(
        [edge_weight, jnp.zeros((pad,), edge_weight.dtype)]).reshape(NW, E_W)
    zeros = jnp.zeros((N_NODES, D), jnp.float32)
    parts = _sc_aggregate(x, src, dst, w, zeros)
    return _tc_finish(parts, W)
